# baseline (device time: 37018 ns/iter reference)
import os

import jax
import jax.numpy as jnp
from jax import lax
from jax.experimental import pallas as pl
from jax.experimental.pallas import tpu as pltpu

_VAR = os.environ.get("KVAR", "")
SKIP_SCATTER = "S" in _VAR
SKIP_ATTN = "T" in _VAR
SKIP_AR = "R" in _VAR
PACK_ONLY = "P" in _VAR
SKIP_BARRIER = "B" in _VAR

N_DEV = 8
B, SQ, DM = 2, 128, 512
HQ, DH = 4, 64
SKV_SHARD = 128
N_CHUNKS = 2
ROWS = SQ // N_DEV
BF16 = jnp.bfloat16

DIRECT = {0: [1, 3, 4], 1: [0, 2, 5]}
REL_BY_ROOT = {
    0: [(2, 1, 0), (5, 1, 1), (7, 3, 0), (6, 4, 0)],
    1: [(3, 0, 0), (4, 0, 1), (6, 2, 0), (7, 5, 0)],
}
RELAYS = [(r, s, j, d) for j, lst in REL_BY_ROOT.items() for d, r, s in lst]
RELAYED = {(j, d) for j, lst in REL_BY_ROOT.items() for d, _, _ in lst}


def kernel(x, Wq, K_ext, V_ext, Wo):
    def body(x_ref, wq_ref, k_ref, v_ref, wo_ref, out_ref,
             kvbuf, relay_buf, partial_ref, rs_buf, red_ref,
             ag_buf,
             kv_send_sems, kv_recv_sems, relay_recv_sems, fwd_sems,
             rs_send_sems, rs_recv_sems,
             ag_send_sems, ag_recv_sems):
        p = lax.axis_index("i")

        def kv_direct_rdma(j, t, d):
            src = k_ref if t == 0 else v_ref
            return pltpu.make_async_remote_copy(
                src_ref=src.at[:, pl.ds(256 * d, 256)],
                dst_ref=kvbuf.at[j, t],
                send_sem=kv_send_sems.at[t, d],
                recv_sem=kv_recv_sems.at[t, j],
                device_id=(d,),
                device_id_type=pl.DeviceIdType.MESH,
            )

        def kv_relay_in_rdma(t, d, rnode, slot):
            src = k_ref if t == 0 else v_ref
            return pltpu.make_async_remote_copy(
                src_ref=src.at[:, pl.ds(256 * d, 256)],
                dst_ref=relay_buf.at[slot, t],
                send_sem=kv_send_sems.at[t, d],
                recv_sem=relay_recv_sems.at[slot, t],
                device_id=(rnode,),
                device_id_type=pl.DeviceIdType.MESH,
            )

        def kv_fwd_rdma(slot, t, j, d):
            return pltpu.make_async_remote_copy(
                src_ref=relay_buf.at[slot, t],
                dst_ref=kvbuf.at[j, t],
                send_sem=fwd_sems.at[slot, t],
                recv_sem=kv_recv_sems.at[t, j],
                device_id=(d,),
                device_id_type=pl.DeviceIdType.MESH,
            )

        if not SKIP_BARRIER:
            bsem = pltpu.get_barrier_semaphore()
            for off in range(1, N_DEV):
                pl.semaphore_signal(
                    bsem, inc=1,
                    device_id=((p + off) % N_DEV,),
                    device_id_type=pl.DeviceIdType.MESH,
                )
            pl.semaphore_wait(bsem, N_DEV - 1)

        for j in range(N_CHUNKS) if not SKIP_SCATTER else []:
            @pl.when(p == j)
            def _(j=j):
                kvbuf[j, 0] = k_ref[:, 256 * j:256 * j + 256]
                kvbuf[j, 1] = v_ref[:, 256 * j:256 * j + 256]
                if not PACK_ONLY:
                    for d, rnode, slot in REL_BY_ROOT[j]:
                        kv_relay_in_rdma(0, d, rnode, slot).start()
                        kv_relay_in_rdma(1, d, rnode, slot).start()
                    for d in DIRECT[j]:
                        kv_direct_rdma(j, 0, d).start()
                        kv_direct_rdma(j, 1, d).start()

        qs = []
        for b in range(B):
            qs.append(jnp.dot(x_ref[b], wq_ref[...],
                              preferred_element_type=jnp.float32)
                      .astype(BF16))

        if not (SKIP_SCATTER or PACK_ONLY):
            for rnode, slot, j, d in RELAYS:
                @pl.when(p == rnode)
                def _(rnode=rnode, slot=slot, j=j, d=d):
                    for t in range(2):
                        kv_relay_in_rdma(t, d, rnode, slot).wait_recv()
                        kv_fwd_rdma(slot, t, j, d).start()

        if not (SKIP_SCATTER or PACK_ONLY):
            for j in range(N_CHUNKS):
                @pl.when(p != j)
                def _(j=j):
                    kv_direct_rdma(j, 0, 0).wait_recv()
                    kv_direct_rdma(j, 1, 0).wait_recv()

        skv = N_CHUNKS * SKV_SHARD
        qi = lax.broadcasted_iota(jnp.int32, (SQ, skv), 0)
        ki = lax.broadcasted_iota(jnp.int32, (SQ, skv), 1)
        mask = jnp.abs(qi - ki) <= 128

        for b in range(B) if not SKIP_ATTN else []:
            cols = []
            for h in range(HQ):
                qbh = qs[b][:, DH * h:DH * h + DH]
                kbh = jnp.concatenate(
                    [kvbuf[0, 0, 128 * b:128 * b + 128, DH * h:DH * h + DH],
                     kvbuf[1, 0, 128 * b:128 * b + 128, DH * h:DH * h + DH]],
                    axis=0)
                vbh = jnp.concatenate(
                    [kvbuf[0, 1, 128 * b:128 * b + 128, DH * h:DH * h + DH],
                     kvbuf[1, 1, 128 * b:128 * b + 128, DH * h:DH * h + DH]],
                    axis=0)
                s = lax.dot_general(
                    qbh, kbh, (((1,), (1,)), ((), ())),
                    preferred_element_type=jnp.float32) * 0.125
                s = jnp.where(mask, s, -1e9)
                m = jnp.max(s, axis=-1, keepdims=True)
                w = jnp.exp(s - m)
                w = (w / jnp.sum(w, axis=-1, keepdims=True)).astype(BF16)
                cols.append(jnp.dot(w, vbh,
                                    preferred_element_type=jnp.float32))
            ctx_b = jnp.concatenate(cols, axis=1).astype(BF16)
            partial_ref[b] = jnp.dot(
                ctx_b, wo_ref[...].astype(BF16),
                preferred_element_type=jnp.float32).astype(BF16)

        if not (SKIP_SCATTER or PACK_ONLY):
            for j in range(N_CHUNKS):
                @pl.when(p == j)
                def _(j=j):
                    for d, rnode, slot in REL_BY_ROOT[j]:
                        kv_relay_in_rdma(0, d, rnode, slot).wait_send()
                        kv_relay_in_rdma(1, d, rnode, slot).wait_send()
                    for d in DIRECT[j]:
                        kv_direct_rdma(j, 0, d).wait_send()
                        kv_direct_rdma(j, 1, d).wait_send()
            for rnode, slot, j, d in RELAYS:
                @pl.when(p == rnode)
                def _(rnode=rnode, slot=slot, j=j, d=d):
                    kv_fwd_rdma(slot, 0, j, d).wait_send()
                    kv_fwd_rdma(slot, 1, j, d).wait_send()

        if SKIP_AR:
            out_ref[...] = partial_ref[...].astype(jnp.float32)
            return

        rs_sends = []
        for off in range(1, N_DEV):
            d = (p + off) % N_DEV
            r = pltpu.make_async_remote_copy(
                src_ref=partial_ref.at[:, pl.ds(ROWS * d, ROWS), :],
                dst_ref=rs_buf.at[p],
                send_sem=rs_send_sems.at[off - 1],
                recv_sem=rs_recv_sems.at[p],
                device_id=(d,),
                device_id_type=pl.DeviceIdType.MESH,
            )
            r.start()
            rs_sends.append(r)

        acc = partial_ref[:, pl.ds(ROWS * p, ROWS), :].astype(jnp.float32)
        for off in range(1, N_DEV):
            src = (p + off) % N_DEV
            r = pltpu.make_async_remote_copy(
                src_ref=partial_ref.at[:, pl.ds(0, ROWS), :],
                dst_ref=rs_buf.at[src],
                send_sem=rs_send_sems.at[0],
                recv_sem=rs_recv_sems.at[src],
                device_id=(src,),
                device_id_type=pl.DeviceIdType.MESH,
            )
            r.wait_recv()
            acc = acc + jnp.squeeze(
                rs_buf[pl.ds(src, 1)], axis=0).astype(jnp.float32)
        out_ref[:, pl.ds(ROWS * p, ROWS), :] = acc
        red_ref[...] = acc.astype(BF16)

        ag_sends = []
        for off in range(1, N_DEV):
            d = (p + off) % N_DEV
            r = pltpu.make_async_remote_copy(
                src_ref=red_ref,
                dst_ref=ag_buf.at[p],
                send_sem=ag_send_sems.at[off - 1],
                recv_sem=ag_recv_sems.at[p],
                device_id=(d,),
                device_id_type=pl.DeviceIdType.MESH,
            )
            r.start()
            ag_sends.append(r)
        for off in range(1, N_DEV):
            src = (p + off) % N_DEV
            r = pltpu.make_async_remote_copy(
                src_ref=red_ref,
                dst_ref=ag_buf.at[src],
                send_sem=ag_send_sems.at[0],
                recv_sem=ag_recv_sems.at[src],
                device_id=(src,),
                device_id_type=pl.DeviceIdType.MESH,
            )
            r.wait_recv()
            out_ref[:, pl.ds(ROWS * src, ROWS), :] = jnp.squeeze(
                ag_buf[pl.ds(src, 1)], axis=0).astype(jnp.float32)

        for r in rs_sends + ag_sends:
            r.wait_send()

    return pl.pallas_call(
        body,
        out_shape=jax.ShapeDtypeStruct((B, SQ, DM), jnp.float32),
        in_specs=[pl.BlockSpec(memory_space=pltpu.VMEM)] * 5,
        out_specs=pl.BlockSpec(memory_space=pltpu.VMEM),
        scratch_shapes=[
            pltpu.VMEM((N_CHUNKS, 2, 256, 256), BF16),
            pltpu.VMEM((2, 2, 256, 256), BF16),
            pltpu.VMEM((B, SQ, DM), BF16),
            pltpu.VMEM((N_DEV, B, ROWS, DM), BF16),
            pltpu.VMEM((B, ROWS, DM), BF16),
            pltpu.VMEM((N_DEV, B, ROWS, DM), BF16),
            pltpu.SemaphoreType.DMA((2, N_DEV)),
            pltpu.SemaphoreType.DMA((2, N_CHUNKS)),
            pltpu.SemaphoreType.DMA((2, 2)),
            pltpu.SemaphoreType.DMA((2, 2)),
            pltpu.SemaphoreType.DMA((N_DEV - 1,)),
            pltpu.SemaphoreType.DMA((N_DEV,)),
            pltpu.SemaphoreType.DMA((N_DEV - 1,)),
            pltpu.SemaphoreType.DMA((N_DEV,)),
        ],
        compiler_params=pltpu.CompilerParams(
            collective_id=None if SKIP_BARRIER else 0),
    )(x, Wq,
      K_ext.astype(BF16).reshape(B * SKV_SHARD, 32 * DH),
      V_ext.astype(BF16).reshape(B * SKV_SHARD, 32 * DH), Wo)
